# baseline (speedup 1.0000x reference)
import jax
import jax.numpy as jnp
from jax import lax
from jax.experimental import pallas as pl
from jax.experimental.pallas import tpu as pltpu

B = 1024
T = 200
D = 128
_UNROLL = 8


def _eos_gather_body(x_hbm, text_ref, out_ref, flat_v, flat_s, sem0, sem1):
    t = text_ref[...]
    cnt = jnp.sum((t != 0).astype(jnp.int32), axis=1)
    eos = jnp.clip(cnt - 1, 0, T - 1)
    flat_v[...] = lax.broadcasted_iota(jnp.int32, (B,), 0) * T + eos

    pltpu.make_async_copy(flat_v, flat_s, sem0).start()
    pltpu.make_async_copy(flat_v, flat_s, sem0).wait()

    def issue(i, carry):
        for u in range(_UNROLL):
            ii = i * _UNROLL + u
            r = flat_s[ii]
            pltpu.make_async_copy(
                x_hbm.at[pl.ds(r, 1)], out_ref.at[pl.ds(ii, 1)], sem1
            ).start(priority=u % 2)
        return carry

    lax.fori_loop(0, B // _UNROLL, issue, 0)
    pltpu.make_async_copy(x_hbm.at[pl.ds(0, B)], out_ref, sem1).wait()


@jax.jit
def kernel(x, text):
    x2 = x.reshape(B * T, D)
    text32 = text.astype(jnp.int32)
    return pl.pallas_call(
        _eos_gather_body,
        in_specs=[
            pl.BlockSpec(memory_space=pl.ANY),
            pl.BlockSpec(memory_space=pltpu.VMEM),
        ],
        out_specs=pl.BlockSpec(memory_space=pl.ANY),
        out_shape=jax.ShapeDtypeStruct((B, D), jnp.float32),
        scratch_shapes=[
            pltpu.VMEM((B,), jnp.int32),
            pltpu.SMEM((B,), jnp.int32),
            pltpu.SemaphoreType.DMA,
            pltpu.SemaphoreType.DMA,
        ],
    )(x2, text32)


# speedup vs baseline: 2.2802x; 2.2802x over previous
import jax
import jax.numpy as jnp
from jax import lax
from jax.experimental import pallas as pl
from jax.experimental.pallas import tpu as pltpu

B = 1024
T = 200
D = 128
_UNROLL = 16


def _eos_gather_body(x_hbm, text_ref, out_ref, flat_v, flat_s, sem0, sem1):
    t = text_ref[...]
    cnt = jnp.sum((t != 0).astype(jnp.int32), axis=1)
    eos = jnp.clip(cnt - 1, 0, T - 1)
    base = lax.broadcasted_iota(jnp.int32, (B,), 0) * T
    flat_v[...] = (base + eos) * D

    pltpu.make_async_copy(flat_v, flat_s, sem0).start()
    pltpu.make_async_copy(flat_v, flat_s, sem0).wait()

    def issue(i, carry):
        for u in range(_UNROLL):
            ii = i * _UNROLL + u
            r = pl.multiple_of(flat_s[ii], D)
            pltpu.make_async_copy(
                x_hbm.at[pl.ds(r, D)], out_ref.at[pl.ds(ii * D, D)], sem1
            ).start(priority=u % 2)
        return carry

    lax.fori_loop(0, B // _UNROLL, issue, 0)
    pltpu.make_async_copy(x_hbm.at[pl.ds(0, B * D)], out_ref, sem1).wait()


@jax.jit
def kernel(x, text):
    x2 = x.reshape(B * T * D)
    text32 = text.astype(jnp.int32)
    out = pl.pallas_call(
        _eos_gather_body,
        in_specs=[
            pl.BlockSpec(memory_space=pl.ANY),
            pl.BlockSpec(memory_space=pltpu.VMEM),
        ],
        out_specs=pl.BlockSpec(memory_space=pltpu.VMEM),
        out_shape=jax.ShapeDtypeStruct((B * D,), jnp.float32),
        scratch_shapes=[
            pltpu.VMEM((B,), jnp.int32),
            pltpu.SMEM((B,), jnp.int32),
            pltpu.SemaphoreType.DMA,
            pltpu.SemaphoreType.DMA,
        ],
    )(x2, text32)
    return out.reshape(B, D)
